# baseline (speedup 1.0000x reference)
import functools

import jax
import jax.numpy as jnp
from jax import lax
from jax.experimental import pallas as pl
from jax.experimental.pallas import tpu as pltpu, tpu_sc as plsc

N = 50000
E = 800000
NF = 10
NI = 9
EF = 11
NO = 9
MS = 96
ITERS = 3

NC = 2
NS = 16
NW = NC * NS

EP = 851968
EPH = EP
G_CHUNK = 1024
GA = 36
GB = 16
S_CHUNK = 1024
S_PW = EPH // NW
S_SUB = S_CHUNK // 128

NP = 50176
ROWS_PER_SUB = NP // NS
TRASH = N

BM = 4096
BN = 6272

@functools.lru_cache(maxsize=None)
def _mesh():
    return plsc.VectorSubcoreMesh(core_axis_name="c", subcore_axis_name="s",
                                  num_cores=NC, num_subcores=NS)


_sc_params = pltpu.CompilerParams(use_tc_tiling_on_sc=False)


def _i32(x):
    return jnp.asarray(x, jnp.int32)


def _wid():
    return (lax.axis_index("s").astype(jnp.int32) * _i32(NC)
            + lax.axis_index("c").astype(jnp.int32))


def _gather_body(h_hbm, src_hbm, dst_hbm, hs_out, hd_out,
                 is_a, is_b, id_a, id_b, rs_a, rs_b, rd_a, rd_b,
                 gsa, gsb, gda, gdb, wsa, wsb, wda, wdb):
    c = lax.axis_index("c").astype(jnp.int32)
    s = lax.axis_index("s").astype(jnp.int32)
    sidx = (is_a, is_b)
    didx = (id_a, id_b)
    srow = (rs_a, rs_b)
    drow = (rd_a, rd_b)
    gs = (gsa, gsb)
    gd = (gda, gdb)
    ws = (wsa, wsb)
    wd = (wda, wdb)

    chunk0 = jnp.where(c == 0, s * _i32(GA),
                       _i32(NS * GA) + s * _i32(GB))
    my_t = jnp.where(c == 0, _i32(GA), _i32(GB))

    def base(j):
        return (chunk0 + j) * _i32(G_CHUNK)

    def start(j, p):
        b = base(j)
        pltpu.sync_copy(src_hbm.at[pl.ds(b, G_CHUNK)], sidx[p])
        pltpu.async_copy(h_hbm.at[sidx[p]], srow[p], gs[p])
        pltpu.sync_copy(dst_hbm.at[pl.ds(b, G_CHUNK)], didx[p])
        pltpu.async_copy(h_hbm.at[didx[p]], drow[p], gd[p])

    def finish(j, p):
        b = pl.ds(base(j), G_CHUNK)
        pltpu.make_async_copy(h_hbm.at[sidx[p]], srow[p], gs[p]).wait()
        pltpu.async_copy(srow[p], hs_out.at[b], ws[p])
        pltpu.make_async_copy(h_hbm.at[didx[p]], drow[p], gd[p]).wait()
        pltpu.async_copy(drow[p], hd_out.at[b], wd[p])

    def drain_wb(p):
        b0 = pl.ds(_i32(0), G_CHUNK)
        pltpu.make_async_copy(srow[p], hs_out.at[b0], ws[p]).wait()
        pltpu.make_async_copy(drow[p], hd_out.at[b0], wd[p]).wait()

    start(_i32(0), 0)
    start(_i32(1), 1)

    def outer(jj, carry):
        j0 = jj * _i32(2)
        finish(j0, 0)
        finish(j0 + _i32(1), 1)
        drain_wb(0)
        start(j0 + _i32(2), 0)
        drain_wb(1)
        start(j0 + _i32(3), 1)
        return carry

    lax.fori_loop(_i32(0), my_t // _i32(2) - _i32(1), outer, _i32(0))
    finish(my_t - _i32(2), 0)
    finish(my_t - _i32(1), 1)
    drain_wb(0)
    drain_wb(1)


@functools.lru_cache(maxsize=None)
def _sc_gather_fn():
    return pl.kernel(
        _gather_body,
        out_type=[jax.ShapeDtypeStruct((EPH, 16), jnp.float32),
                  jax.ShapeDtypeStruct((EPH, 16), jnp.float32)],
        mesh=_mesh(),
        scratch_types=[pltpu.VMEM((G_CHUNK,), jnp.int32)] * 4
        + [pltpu.VMEM((G_CHUNK, 16), jnp.float32)] * 4
        + [pltpu.SemaphoreType.DMA] * 8,
        compiler_params=_sc_params,
    )


def _sc_gather(h16, src, dst):
    return _sc_gather_fn()(h16, src, dst)


def _scatter_core(dst2_hbm, z_hbm, out_hbm, idx2, agg_s, asem, load, src,
                  s_outer):
    c = lax.axis_index("c").astype(jnp.int32)
    s = lax.axis_index("s").astype(jnp.int32)
    w = _wid()
    pltpu.sync_copy(z_hbm,
                    agg_s.at[pl.ds(s * _i32(ROWS_PER_SUB), ROWS_PER_SUB)])
    plsc.subcore_barrier()

    def loads(t, p):
        base128 = w * _i32(s_outer * S_SUB) + t * _i32(S_SUB)
        pltpu.sync_copy(dst2_hbm.at[pl.ds(base128, S_SUB)], idx2[p])
        load(t, p)

    def adds(p):
        return [pltpu.async_copy(src(p, k), agg_s.at[idx2[p].at[_i32(k)]],
                                 asem[p], add=True)
                for k in range(S_SUB)]

    peel = s_outer % 2
    if peel:
        loads(_i32(0), 0)
        for d in adds(0):
            d.wait()

    def outer(jj, carry):
        t0 = _i32(peel) + jj * _i32(2)
        loads(t0, 0)
        ad0 = adds(0)
        loads(t0 + _i32(1), 1)
        ad1 = adds(1)
        for d in ad0 + ad1:
            d.wait()
        return carry

    lax.fori_loop(_i32(0), _i32((s_outer - peel) // 2), outer, _i32(0))
    plsc.subcore_barrier()
    pltpu.sync_copy(
        agg_s.at[pl.ds(s * _i32(ROWS_PER_SUB), ROWS_PER_SUB)],
        out_hbm.at[c, pl.ds(s * _i32(ROWS_PER_SUB), ROWS_PER_SUB)])


def _scatter_body(dst2_hbm, msg_hbm, z_hbm, out_hbm, idx_a, idx_b,
                  msg_a, msg_b, agg_s, as_a, as_b):
    w = _wid()
    msgs = (msg_a, msg_b)
    s_outer = S_PW // S_CHUNK

    def load(t, p):
        base = w * _i32(S_PW) + t * _i32(S_CHUNK)
        pltpu.sync_copy(msg_hbm.at[pl.ds(base, S_CHUNK)], msgs[p])

    def src(p, k):
        return msgs[p].at[pl.ds(_i32(k * 128), 128)]

    _scatter_core(dst2_hbm, z_hbm, out_hbm, (idx_a, idx_b), agg_s,
                  (as_a, as_b), load, src, s_outer)


def _scatter_const_body(dst2_hbm, m0_hbm, z_hbm, out_hbm, idx_a, idx_b,
                        m0_v, agg_s, as_a, as_b):
    pltpu.sync_copy(m0_hbm, m0_v)

    def load(t, p):
        pass

    def src(p, k):
        return m0_v

    _scatter_core(dst2_hbm, z_hbm, out_hbm, (idx_a, idx_b), agg_s,
                  (as_a, as_b), load, src, EP // NW // S_CHUNK)


_agg_out = jax.ShapeDtypeStruct((NC, NP, 16), jnp.float32)


@functools.lru_cache(maxsize=None)
def _sc_scatter_fn():
    return pl.kernel(
        _scatter_body, out_type=_agg_out, mesh=_mesh(),
        scratch_types=[pltpu.VMEM((S_SUB, 128), jnp.int32),
                       pltpu.VMEM((S_SUB, 128), jnp.int32),
                       pltpu.VMEM((S_CHUNK, 16), jnp.float32),
                       pltpu.VMEM((S_CHUNK, 16), jnp.float32),
                       pltpu.VMEM_SHARED((NP, 16), jnp.float32),
                       pltpu.SemaphoreType.DMA,
                       pltpu.SemaphoreType.DMA],
        compiler_params=_sc_params)


@functools.lru_cache(maxsize=None)
def _sc_scatter_const_fn():
    return pl.kernel(
        _scatter_const_body, out_type=_agg_out, mesh=_mesh(),
        scratch_types=[pltpu.VMEM((S_SUB, 128), jnp.int32),
                       pltpu.VMEM((S_SUB, 128), jnp.int32),
                       pltpu.VMEM((128, 16), jnp.float32),
                       pltpu.VMEM_SHARED((NP, 16), jnp.float32),
                       pltpu.SemaphoreType.DMA,
                       pltpu.SemaphoreType.DMA],
        compiler_params=_sc_params)


def _sc_scatter(dst2h, msgh, zrows):
    return _sc_scatter_fn()(dst2h, msgh, zrows)


def _sc_scatter_const(dst2, m0t, zrows):
    return _sc_scatter_const_fn()(dst2, m0t, zrows)


BMR = BM // 8


def _mlp_body(hs, hd, a1, b1, w2, w3, w4s, m1, m2, m3, m4, out):
    f32 = jnp.float32
    bf = jnp.bfloat16

    def dot(x, w):
        return jax.lax.dot(x.astype(bf), w, precision=None,
                           preferred_element_type=f32)

    x1 = dot(hs[...], a1[...]) + dot(hd[...], b1[...]) + m1[...][0:1]
    x1 = jnp.maximum(x1, 0.0)
    x2 = jnp.concatenate(
        [jnp.maximum(dot(x1[:, 128 * k:128 * (k + 1)], w2[...]) + m2[...][0:1],
                     0.0) for k in range(8)], axis=1)
    x3 = jnp.concatenate(
        [jnp.maximum(dot(x2[:, 128 * k:128 * (k + 1)], w3[...]) + m3[...][0:1],
                     0.0) for k in range(8)], axis=1)
    acc = m4[...][0:1] + jnp.zeros((BMR, 128), f32)
    for k in range(8):
        acc = acc + dot(x3[:, 128 * k:128 * (k + 1)],
                        w4s[...][128 * k:128 * (k + 1), :])
    out[...] = acc


def _full(shape):
    return pl.BlockSpec(shape, lambda i: (jnp.int32(0), jnp.int32(0)))


def _tc_mlp(hs, hd, a1, b1, w2, w3, w4s, m1, m2, m3, m4):
    grid = EPH // BM
    blk = pl.BlockSpec((BMR, 128), lambda i: (i, jnp.int32(0)))
    return pl.pallas_call(
        _mlp_body,
        grid=(grid,),
        in_specs=[blk, blk, _full((128, 1024)), _full((128, 1024)),
                  _full((128, 128)), _full((128, 128)), _full((1024, 128)),
                  _full((8, 1024)), _full((8, 128)), _full((8, 128)),
                  _full((8, 128))],
        out_specs=blk,
        out_shape=jax.ShapeDtypeStruct((EPH // 8, 128), jnp.float32),
        compiler_params=pltpu.CompilerParams(
            dimension_semantics=("arbitrary",)),
    )(hs, hd, a1, b1, w2, w3, w4s, m1, m2, m3, m4)


def _gru_body(a0, a1, a2, a3, ni, h, war, wnr, whr, waz, wnz, whz, wan, wnn,
              whn, br, bz, bn, bhr, bhz, bhn, fwp, fbp, hout, oout):
    agg = (a0[...] + a1[...]) + (a2[...] + a3[...])
    nn = ni[...]
    hh = h[...]
    i_r = jnp.dot(agg, war[...]) + jnp.dot(nn, wnr[...]) + br[...][0:1]
    i_z = jnp.dot(agg, waz[...]) + jnp.dot(nn, wnz[...]) + bz[...][0:1]
    i_n = jnp.dot(agg, wan[...]) + jnp.dot(nn, wnn[...]) + bn[...][0:1]
    h_r = jnp.dot(hh, whr[...]) + bhr[...][0:1]
    h_z = jnp.dot(hh, whz[...]) + bhz[...][0:1]
    h_n = jnp.dot(hh, whn[...]) + bhn[...][0:1]
    r = jax.nn.sigmoid(i_r + h_r)
    z = jax.nn.sigmoid(i_z + h_z)
    n = jnp.tanh(i_n + r * h_n)
    hnew = (1.0 - z) * n + z * hh
    hout[...] = hnew
    oout[...] = jnp.dot(hnew, fwp[...]) + fbp[...][0:1]


BNR = BN // 8


def _tc_gru(a0, a1, a2, a3, ni, h, ws, bs, fwp, fbp):
    grid = NP // BN
    blk = pl.BlockSpec((BNR, 128), lambda i: (i, jnp.int32(0)))
    wspec = _full((128, 128))
    bspec = _full((8, 128))
    return pl.pallas_call(
        _gru_body,
        grid=(grid,),
        in_specs=[blk, blk, blk, blk, blk, blk] + [wspec] * 9 + [bspec] * 6
        + [wspec, bspec],
        out_specs=[blk, blk],
        out_shape=[jax.ShapeDtypeStruct((NP // 8, 128), jnp.float32),
                   jax.ShapeDtypeStruct((NP // 8, 128), jnp.float32)],
        compiler_params=pltpu.CompilerParams(
            dimension_semantics=("arbitrary",)),
    )(a0, a1, a2, a3, ni, h, *ws, *bs, fwp, fbp)


def kernel(node_inputs, src_ids, dst_ids, mw1, mb1, mw2, mb2, mw3, mb3, mw4,
           mb4, w_ih, w_hh, b_ih, b_hh, fw, fb):
    f32 = jnp.float32
    epad = EP - E
    src = jnp.concatenate([src_ids.astype(jnp.int32),
                           jnp.zeros((epad,), jnp.int32)])
    dst = jnp.concatenate([dst_ids.astype(jnp.int32),
                           jnp.full((epad,), TRASH, jnp.int32)])
    dst2 = dst.reshape(EP // 128, 128)

    ni128 = jnp.zeros((NP, 16), f32).at[:N, :NI].set(
        node_inputs.astype(f32)).reshape(NP // 8, 128)

    eye8 = jnp.eye(8, dtype=f32)

    def brow(vec16):
        return jnp.tile(jnp.tile(vec16, 8)[None, :], (8, 1))

    bf = jnp.bfloat16
    a1p = jnp.zeros((16, 128), f32).at[:NF, :MS].set(mw1[:, :NF].T)
    b1p = jnp.zeros((16, 128), f32).at[:NF, :MS].set(mw1[:, NF:].T)
    a1 = jnp.kron(eye8, a1p).astype(bf)
    b1 = jnp.kron(eye8, b1p).astype(bf)
    w2 = jnp.zeros((128, 128), f32).at[:MS, :MS].set(mw2.T).astype(bf)
    w3 = jnp.zeros((128, 128), f32).at[:MS, :MS].set(mw3.T).astype(bf)
    w4s = jnp.concatenate(
        [jnp.zeros((128, 128), f32).at[:MS, 16 * k:16 * k + EF].set(mw4.T)
         for k in range(8)], axis=0).astype(bf)
    m1 = jnp.tile(jnp.tile(jnp.zeros(128, f32).at[:MS].set(mb1), 8)[None, :],
                  (8, 1))
    m2 = jnp.tile(jnp.zeros(128, f32).at[:MS].set(mb2)[None, :], (8, 1))
    m3 = jnp.tile(jnp.zeros(128, f32).at[:MS].set(mb3)[None, :], (8, 1))
    m4 = brow(jnp.zeros(16, f32).at[:EF].set(mb4))

    def gw(mat, g, in_lo, in_hi, in_n):
        out = jnp.zeros((16, 16), f32)
        out = out.at[:in_n, :NF].set(mat[NF * g:NF * (g + 1), in_lo:in_hi].T)
        return jnp.kron(eye8, out)

    ws = []
    for g in range(3):
        ws.append(gw(w_ih, g, 0, EF, EF))
        ws.append(gw(w_ih, g, EF, EF + NI, NI))
        ws.append(gw(w_hh, g, 0, NF, NF))

    def gb(vec, g):
        return brow(jnp.zeros(16, f32).at[:NF].set(vec[NF * g:NF * (g + 1)]))

    bs = [gb(b_ih, 0), gb(b_ih, 1), gb(b_ih, 2),
          gb(b_hh, 0), gb(b_hh, 1), gb(b_hh, 2)]

    fwp = jnp.kron(eye8, jnp.zeros((16, 16), f32).at[:NF, :NO].set(fw.T))
    fbp = brow(jnp.zeros(16, f32).at[:NO].set(fb))

    h1_ = jnp.maximum(mb1[None, :], 0.0)
    h2_ = jnp.maximum(jnp.dot(h1_, mw2.T) + mb2[None, :], 0.0)
    h3_ = jnp.maximum(jnp.dot(h2_, mw3.T) + mb3[None, :], 0.0)
    m0 = jnp.dot(h3_, mw4.T) + mb4[None, :]
    m0t = jnp.zeros((128, 16), f32).at[:, :EF].set(m0)

    zrows = jnp.zeros((ROWS_PER_SUB, 16), f32)
    h128 = jnp.zeros((NP // 8, 128), f32)
    zagg = jnp.zeros((NP // 8, 128), f32)

    outs = []
    agg0 = _sc_scatter_const(dst2, m0t, zrows).reshape(NC, NP // 8, 128)
    aggs = (agg0[0], agg0[1], zagg, zagg)
    for it in range(ITERS):
        h128, o = _tc_gru(*aggs, ni128, h128, ws, bs, fwp, fbp)
        outs.append(o.reshape(NP, 16)[:N, :NO])
        if it < ITERS - 1:
            h16sc = h128.reshape(NP, 16)
            hs, hd = _sc_gather(h16sc, src, dst)
            msg = _tc_mlp(hs.reshape(EPH // 8, 128),
                          hd.reshape(EPH // 8, 128),
                          a1, b1, w2, w3, w4s, m1, m2, m3, m4)
            ap = _sc_scatter(dst2, msg.reshape(EPH, 16),
                             zrows).reshape(NC, NP // 8, 128)
            aggs = (ap[0], ap[1], zagg, zagg)
    return jnp.stack(outs, axis=0)


# speedup vs baseline: 1.2096x; 1.2096x over previous
import functools

import jax
import jax.numpy as jnp
from jax import lax
from jax.experimental import pallas as pl
from jax.experimental.pallas import tpu as pltpu, tpu_sc as plsc

N = 50000
E = 800000
NF = 10
NI = 9
EF = 11
NO = 9
MS = 96
ITERS = 3

NC = 2
NS = 16
NW = NC * NS

EP = 819200
EPH = EP
G_CHUNK = 1600
GA = 22
GB = 10
S_CHUNK = 1024
S_PW = EPH // NW
S_SUB = S_CHUNK // 128

NP = 50176
ROWS_PER_SUB = NP // NS
TRASH = N

BM = 4096
BN = 6272

@functools.lru_cache(maxsize=None)
def _mesh():
    return plsc.VectorSubcoreMesh(core_axis_name="c", subcore_axis_name="s",
                                  num_cores=NC, num_subcores=NS)


_sc_params = pltpu.CompilerParams(use_tc_tiling_on_sc=False)


def _i32(x):
    return jnp.asarray(x, jnp.int32)


def _wid():
    return (lax.axis_index("s").astype(jnp.int32) * _i32(NC)
            + lax.axis_index("c").astype(jnp.int32))


def _gather_body(h_hbm, src_hbm, dst_hbm, hs_out, hd_out,
                 is_a, is_b, id_a, id_b, rs_a, rs_b, rd_a, rd_b,
                 gsa, gsb, gda, gdb, wsa, wsb, wda, wdb):
    c = lax.axis_index("c").astype(jnp.int32)
    s = lax.axis_index("s").astype(jnp.int32)
    sidx = (is_a, is_b)
    didx = (id_a, id_b)
    srow = (rs_a, rs_b)
    drow = (rd_a, rd_b)
    gs = (gsa, gsb)
    gd = (gda, gdb)
    ws = (wsa, wsb)
    wd = (wda, wdb)

    chunk0 = jnp.where(c == 0, s * _i32(GA),
                       _i32(NS * GA) + s * _i32(GB))
    my_t = jnp.where(c == 0, _i32(GA), _i32(GB))

    def base(j):
        return (chunk0 + j) * _i32(G_CHUNK)

    def start(j, p):
        b = base(j)
        pltpu.sync_copy(src_hbm.at[pl.ds(b, G_CHUNK)], sidx[p])
        pltpu.async_copy(h_hbm.at[sidx[p]], srow[p], gs[p])
        pltpu.sync_copy(dst_hbm.at[pl.ds(b, G_CHUNK)], didx[p])
        pltpu.async_copy(h_hbm.at[didx[p]], drow[p], gd[p])

    def finish(j, p):
        b = pl.ds(base(j), G_CHUNK)
        pltpu.make_async_copy(h_hbm.at[sidx[p]], srow[p], gs[p]).wait()
        pltpu.async_copy(srow[p], hs_out.at[b], ws[p])
        pltpu.make_async_copy(h_hbm.at[didx[p]], drow[p], gd[p]).wait()
        pltpu.async_copy(drow[p], hd_out.at[b], wd[p])

    def drain_wb(p):
        b0 = pl.ds(_i32(0), G_CHUNK)
        pltpu.make_async_copy(srow[p], hs_out.at[b0], ws[p]).wait()
        pltpu.make_async_copy(drow[p], hd_out.at[b0], wd[p]).wait()

    start(_i32(0), 0)
    start(_i32(1), 1)

    def outer(jj, carry):
        j0 = jj * _i32(2)
        finish(j0, 0)
        finish(j0 + _i32(1), 1)
        drain_wb(0)
        start(j0 + _i32(2), 0)
        drain_wb(1)
        start(j0 + _i32(3), 1)
        return carry

    lax.fori_loop(_i32(0), my_t // _i32(2) - _i32(1), outer, _i32(0))
    finish(my_t - _i32(2), 0)
    finish(my_t - _i32(1), 1)
    drain_wb(0)
    drain_wb(1)


@functools.lru_cache(maxsize=None)
def _sc_gather_fn():
    return pl.kernel(
        _gather_body,
        out_type=[jax.ShapeDtypeStruct((EPH, 16), jnp.float32),
                  jax.ShapeDtypeStruct((EPH, 16), jnp.float32)],
        mesh=_mesh(),
        scratch_types=[pltpu.VMEM((G_CHUNK,), jnp.int32)] * 4
        + [pltpu.VMEM((G_CHUNK, 16), jnp.float32)] * 4
        + [pltpu.SemaphoreType.DMA] * 8,
        compiler_params=_sc_params,
    )


def _sc_gather(h16, src, dst):
    return _sc_gather_fn()(h16, src, dst)


def _scatter_core(dst2_hbm, z_hbm, out_hbm, idx2, agg_s, asem, load, src,
                  s_outer):
    c = lax.axis_index("c").astype(jnp.int32)
    s = lax.axis_index("s").astype(jnp.int32)
    w = _wid()
    pltpu.sync_copy(z_hbm,
                    agg_s.at[pl.ds(s * _i32(ROWS_PER_SUB), ROWS_PER_SUB)])
    plsc.subcore_barrier()

    def loads(t, p):
        base128 = w * _i32(s_outer * S_SUB) + t * _i32(S_SUB)
        pltpu.sync_copy(dst2_hbm.at[pl.ds(base128, S_SUB)], idx2[p])
        load(t, p)

    def adds(p):
        return [pltpu.async_copy(src(p, k), agg_s.at[idx2[p].at[_i32(k)]],
                                 asem[p], add=True)
                for k in range(S_SUB)]

    peel = s_outer % 2
    if peel:
        loads(_i32(0), 0)
        for d in adds(0):
            d.wait()

    def outer(jj, carry):
        t0 = _i32(peel) + jj * _i32(2)
        loads(t0, 0)
        ad0 = adds(0)
        loads(t0 + _i32(1), 1)
        ad1 = adds(1)
        for d in ad0 + ad1:
            d.wait()
        return carry

    lax.fori_loop(_i32(0), _i32((s_outer - peel) // 2), outer, _i32(0))
    plsc.subcore_barrier()
    pltpu.sync_copy(
        agg_s.at[pl.ds(s * _i32(ROWS_PER_SUB), ROWS_PER_SUB)],
        out_hbm.at[c, pl.ds(s * _i32(ROWS_PER_SUB), ROWS_PER_SUB)])


def _scatter_body(dst2_hbm, msg_hbm, z_hbm, out_hbm, idx_a, idx_b,
                  msg_a, msg_b, agg_s, as_a, as_b):
    w = _wid()
    msgs = (msg_a, msg_b)
    s_outer = S_PW // S_CHUNK

    def load(t, p):
        base = w * _i32(S_PW) + t * _i32(S_CHUNK)
        pltpu.sync_copy(msg_hbm.at[pl.ds(base, S_CHUNK)], msgs[p])

    def src(p, k):
        return msgs[p].at[pl.ds(_i32(k * 128), 128)]

    _scatter_core(dst2_hbm, z_hbm, out_hbm, (idx_a, idx_b), agg_s,
                  (as_a, as_b), load, src, s_outer)


def _scatter_const_body(dst2_hbm, m0_hbm, z_hbm, out_hbm, idx_a, idx_b,
                        m0_v, agg_s, as_a, as_b):
    pltpu.sync_copy(m0_hbm, m0_v)

    def load(t, p):
        pass

    def src(p, k):
        return m0_v

    _scatter_core(dst2_hbm, z_hbm, out_hbm, (idx_a, idx_b), agg_s,
                  (as_a, as_b), load, src, EP // NW // S_CHUNK)


_agg_out = jax.ShapeDtypeStruct((NC, NP, 16), jnp.float32)


@functools.lru_cache(maxsize=None)
def _sc_scatter_fn():
    return pl.kernel(
        _scatter_body, out_type=_agg_out, mesh=_mesh(),
        scratch_types=[pltpu.VMEM((S_SUB, 128), jnp.int32),
                       pltpu.VMEM((S_SUB, 128), jnp.int32),
                       pltpu.VMEM((S_CHUNK, 16), jnp.float32),
                       pltpu.VMEM((S_CHUNK, 16), jnp.float32),
                       pltpu.VMEM_SHARED((NP, 16), jnp.float32),
                       pltpu.SemaphoreType.DMA,
                       pltpu.SemaphoreType.DMA],
        compiler_params=_sc_params)


@functools.lru_cache(maxsize=None)
def _sc_scatter_const_fn():
    return pl.kernel(
        _scatter_const_body, out_type=_agg_out, mesh=_mesh(),
        scratch_types=[pltpu.VMEM((S_SUB, 128), jnp.int32),
                       pltpu.VMEM((S_SUB, 128), jnp.int32),
                       pltpu.VMEM((128, 16), jnp.float32),
                       pltpu.VMEM_SHARED((NP, 16), jnp.float32),
                       pltpu.SemaphoreType.DMA,
                       pltpu.SemaphoreType.DMA],
        compiler_params=_sc_params)


def _sc_scatter(dst2h, msgh, zrows):
    return _sc_scatter_fn()(dst2h, msgh, zrows)


def _sc_scatter_const(dst2, m0t, zrows):
    return _sc_scatter_const_fn()(dst2, m0t, zrows)


BMR = BM // 8


def _mlp_body(hs, hd, a1, b1, w2, w3, w4s, m1, m2, m3, m4, out):
    f32 = jnp.float32
    bf = jnp.bfloat16

    def dot(x, w):
        return jax.lax.dot(x.astype(bf), w, precision=None,
                           preferred_element_type=f32)

    x1 = dot(hs[...], a1[...]) + dot(hd[...], b1[...]) + m1[...][0:1]
    x1 = jnp.maximum(x1, 0.0)
    x2 = jnp.concatenate(
        [jnp.maximum(dot(x1[:, 128 * k:128 * (k + 1)], w2[...]) + m2[...][0:1],
                     0.0) for k in range(8)], axis=1)
    x3 = jnp.concatenate(
        [jnp.maximum(dot(x2[:, 128 * k:128 * (k + 1)], w3[...]) + m3[...][0:1],
                     0.0) for k in range(8)], axis=1)
    acc = m4[...][0:1] + jnp.zeros((BMR, 128), f32)
    for k in range(8):
        acc = acc + dot(x3[:, 128 * k:128 * (k + 1)],
                        w4s[...][128 * k:128 * (k + 1), :])
    out[...] = acc


def _full(shape):
    return pl.BlockSpec(shape, lambda i: (jnp.int32(0), jnp.int32(0)))


def _tc_mlp(hs, hd, a1, b1, w2, w3, w4s, m1, m2, m3, m4):
    grid = EPH // BM
    blk = pl.BlockSpec((BMR, 128), lambda i: (i, jnp.int32(0)))
    return pl.pallas_call(
        _mlp_body,
        grid=(grid,),
        in_specs=[blk, blk, _full((128, 1024)), _full((128, 1024)),
                  _full((128, 128)), _full((128, 128)), _full((1024, 128)),
                  _full((8, 1024)), _full((8, 128)), _full((8, 128)),
                  _full((8, 128))],
        out_specs=blk,
        out_shape=jax.ShapeDtypeStruct((EPH // 8, 128), jnp.float32),
        compiler_params=pltpu.CompilerParams(
            dimension_semantics=("arbitrary",)),
    )(hs, hd, a1, b1, w2, w3, w4s, m1, m2, m3, m4)


def _gru_body(a0, a1, ni, h, war, wnr, whr, waz, wnz, whz, wan, wnn,
              whn, br, bz, bn, bhr, bhz, bhn, fwp, fbp, hout, oout):
    agg = a0[...] + a1[...]
    nn = ni[...]
    hh = h[...]
    i_r = jnp.dot(agg, war[...]) + jnp.dot(nn, wnr[...]) + br[...][0:1]
    i_z = jnp.dot(agg, waz[...]) + jnp.dot(nn, wnz[...]) + bz[...][0:1]
    i_n = jnp.dot(agg, wan[...]) + jnp.dot(nn, wnn[...]) + bn[...][0:1]
    h_r = jnp.dot(hh, whr[...]) + bhr[...][0:1]
    h_z = jnp.dot(hh, whz[...]) + bhz[...][0:1]
    h_n = jnp.dot(hh, whn[...]) + bhn[...][0:1]
    r = jax.nn.sigmoid(i_r + h_r)
    z = jax.nn.sigmoid(i_z + h_z)
    n = jnp.tanh(i_n + r * h_n)
    hnew = (1.0 - z) * n + z * hh
    hout[...] = hnew
    oout[...] = jnp.dot(hnew, fwp[...]) + fbp[...][0:1]


BNR = BN // 8


def _tc_gru(a0, a1, ni, h, ws, bs, fwp, fbp):
    grid = NP // BN
    blk = pl.BlockSpec((BNR, 128), lambda i: (i, jnp.int32(0)))
    wspec = _full((128, 128))
    bspec = _full((8, 128))
    return pl.pallas_call(
        _gru_body,
        grid=(grid,),
        in_specs=[blk, blk, blk, blk] + [wspec] * 9 + [bspec] * 6
        + [wspec, bspec],
        out_specs=[blk, blk],
        out_shape=[jax.ShapeDtypeStruct((NP // 8, 128), jnp.float32),
                   jax.ShapeDtypeStruct((NP // 8, 128), jnp.float32)],
        compiler_params=pltpu.CompilerParams(
            dimension_semantics=("arbitrary",)),
    )(a0, a1, ni, h, *ws, *bs, fwp, fbp)


def kernel(node_inputs, src_ids, dst_ids, mw1, mb1, mw2, mb2, mw3, mb3, mw4,
           mb4, w_ih, w_hh, b_ih, b_hh, fw, fb):
    f32 = jnp.float32
    epad = EP - E
    src = jnp.concatenate([src_ids.astype(jnp.int32),
                           jnp.zeros((epad,), jnp.int32)])
    dst = jnp.concatenate([dst_ids.astype(jnp.int32),
                           jnp.full((epad,), TRASH, jnp.int32)])
    dst2 = dst.reshape(EP // 128, 128)

    ni128 = jnp.zeros((NP, 16), f32).at[:N, :NI].set(
        node_inputs.astype(f32)).reshape(NP // 8, 128)

    eye8 = jnp.eye(8, dtype=f32)

    def brow(vec16):
        return jnp.tile(jnp.tile(vec16, 8)[None, :], (8, 1))

    bf = jnp.bfloat16
    a1p = jnp.zeros((16, 128), f32).at[:NF, :MS].set(mw1[:, :NF].T)
    b1p = jnp.zeros((16, 128), f32).at[:NF, :MS].set(mw1[:, NF:].T)
    a1 = jnp.kron(eye8, a1p).astype(bf)
    b1 = jnp.kron(eye8, b1p).astype(bf)
    w2 = jnp.zeros((128, 128), f32).at[:MS, :MS].set(mw2.T).astype(bf)
    w3 = jnp.zeros((128, 128), f32).at[:MS, :MS].set(mw3.T).astype(bf)
    w4s = jnp.concatenate(
        [jnp.zeros((128, 128), f32).at[:MS, 16 * k:16 * k + EF].set(mw4.T)
         for k in range(8)], axis=0).astype(bf)
    m1 = jnp.tile(jnp.tile(jnp.zeros(128, f32).at[:MS].set(mb1), 8)[None, :],
                  (8, 1))
    m2 = jnp.tile(jnp.zeros(128, f32).at[:MS].set(mb2)[None, :], (8, 1))
    m3 = jnp.tile(jnp.zeros(128, f32).at[:MS].set(mb3)[None, :], (8, 1))
    m4 = brow(jnp.zeros(16, f32).at[:EF].set(mb4))

    def gw(mat, g, in_lo, in_hi, in_n):
        out = jnp.zeros((16, 16), f32)
        out = out.at[:in_n, :NF].set(mat[NF * g:NF * (g + 1), in_lo:in_hi].T)
        return jnp.kron(eye8, out)

    ws = []
    for g in range(3):
        ws.append(gw(w_ih, g, 0, EF, EF))
        ws.append(gw(w_ih, g, EF, EF + NI, NI))
        ws.append(gw(w_hh, g, 0, NF, NF))

    def gb(vec, g):
        return brow(jnp.zeros(16, f32).at[:NF].set(vec[NF * g:NF * (g + 1)]))

    bs = [gb(b_ih, 0), gb(b_ih, 1), gb(b_ih, 2),
          gb(b_hh, 0), gb(b_hh, 1), gb(b_hh, 2)]

    fwp = jnp.kron(eye8, jnp.zeros((16, 16), f32).at[:NF, :NO].set(fw.T))
    fbp = brow(jnp.zeros(16, f32).at[:NO].set(fb))

    h1_ = jnp.maximum(mb1[None, :], 0.0)
    h2_ = jnp.maximum(jnp.dot(h1_, mw2.T) + mb2[None, :], 0.0)
    h3_ = jnp.maximum(jnp.dot(h2_, mw3.T) + mb3[None, :], 0.0)
    m0 = jnp.dot(h3_, mw4.T) + mb4[None, :]
    m0t = jnp.zeros((128, 16), f32).at[:, :EF].set(m0)

    zrows = jnp.zeros((ROWS_PER_SUB, 16), f32)
    h128 = jnp.zeros((NP // 8, 128), f32)
    zagg = jnp.zeros((NP // 8, 128), f32)

    outs = []
    agg0 = _sc_scatter_const(dst2, m0t, zrows).reshape(NC, NP // 8, 128)
    aggs = (agg0[0], agg0[1])
    for it in range(ITERS):
        h128, o = _tc_gru(*aggs, ni128, h128, ws, bs, fwp, fbp)
        outs.append(o.reshape(NP, 16)[:N, :NO])
        if it < ITERS - 1:
            h16sc = h128.reshape(NP, 16)
            hs, hd = _sc_gather(h16sc, src, dst)
            msg = _tc_mlp(hs.reshape(EPH // 8, 128),
                          hd.reshape(EPH // 8, 128),
                          a1, b1, w2, w3, w4s, m1, m2, m3, m4)
            ap = _sc_scatter(dst2, msg.reshape(EPH, 16),
                             zrows).reshape(NC, NP // 8, 128)
            aggs = (ap[0], ap[1])
    return jnp.stack(outs, axis=0)


# speedup vs baseline: 1.2523x; 1.0353x over previous
import functools

import jax
import jax.numpy as jnp
from jax import lax
from jax.experimental import pallas as pl
from jax.experimental.pallas import tpu as pltpu, tpu_sc as plsc

N = 50000
E = 800000
NF = 10
NI = 9
EF = 11
NO = 9
MS = 96
ITERS = 3

NC = 2
NS = 16
NW = NC * NS

EP = 819200
EPH = EP
G_CHUNK = 1600
GA = 22
GB = 10
S_CHUNK = 1024
S_PW = EPH // NW
S_SUB = S_CHUNK // 128

NP = 50176
ROWS_PER_SUB = NP // NS
TRASH = N

BM = 8192
BN = 6272

@functools.lru_cache(maxsize=None)
def _mesh():
    return plsc.VectorSubcoreMesh(core_axis_name="c", subcore_axis_name="s",
                                  num_cores=NC, num_subcores=NS)


_sc_params = pltpu.CompilerParams(use_tc_tiling_on_sc=False)


def _i32(x):
    return jnp.asarray(x, jnp.int32)


def _wid():
    return (lax.axis_index("s").astype(jnp.int32) * _i32(NC)
            + lax.axis_index("c").astype(jnp.int32))


def _gather_body(h_hbm, src_hbm, dst_hbm, hs_out, hd_out,
                 is_a, is_b, id_a, id_b, rs_a, rs_b, rd_a, rd_b,
                 gsa, gsb, gda, gdb, wsa, wsb, wda, wdb):
    c = lax.axis_index("c").astype(jnp.int32)
    s = lax.axis_index("s").astype(jnp.int32)
    sidx = (is_a, is_b)
    didx = (id_a, id_b)
    srow = (rs_a, rs_b)
    drow = (rd_a, rd_b)
    gs = (gsa, gsb)
    gd = (gda, gdb)
    ws = (wsa, wsb)
    wd = (wda, wdb)

    chunk0 = jnp.where(c == 0, s * _i32(GA),
                       _i32(NS * GA) + s * _i32(GB))
    my_t = jnp.where(c == 0, _i32(GA), _i32(GB))

    def base(j):
        return (chunk0 + j) * _i32(G_CHUNK)

    def start(j, p):
        b = base(j)
        pltpu.sync_copy(src_hbm.at[pl.ds(b, G_CHUNK)], sidx[p])
        pltpu.async_copy(h_hbm.at[sidx[p]], srow[p], gs[p])
        pltpu.sync_copy(dst_hbm.at[pl.ds(b, G_CHUNK)], didx[p])
        pltpu.async_copy(h_hbm.at[didx[p]], drow[p], gd[p])

    def finish(j, p):
        b = pl.ds(base(j), G_CHUNK)
        pltpu.make_async_copy(h_hbm.at[sidx[p]], srow[p], gs[p]).wait()
        pltpu.async_copy(srow[p], hs_out.at[b], ws[p])
        pltpu.make_async_copy(h_hbm.at[didx[p]], drow[p], gd[p]).wait()
        pltpu.async_copy(drow[p], hd_out.at[b], wd[p])

    def drain_wb(p):
        b0 = pl.ds(_i32(0), G_CHUNK)
        pltpu.make_async_copy(srow[p], hs_out.at[b0], ws[p]).wait()
        pltpu.make_async_copy(drow[p], hd_out.at[b0], wd[p]).wait()

    start(_i32(0), 0)
    start(_i32(1), 1)

    def outer(jj, carry):
        j0 = jj * _i32(2)
        finish(j0, 0)
        finish(j0 + _i32(1), 1)
        drain_wb(0)
        start(j0 + _i32(2), 0)
        drain_wb(1)
        start(j0 + _i32(3), 1)
        return carry

    lax.fori_loop(_i32(0), my_t // _i32(2) - _i32(1), outer, _i32(0))
    finish(my_t - _i32(2), 0)
    finish(my_t - _i32(1), 1)
    drain_wb(0)
    drain_wb(1)


@functools.lru_cache(maxsize=None)
def _sc_gather_fn():
    return pl.kernel(
        _gather_body,
        out_type=[jax.ShapeDtypeStruct((EPH, 16), jnp.float32),
                  jax.ShapeDtypeStruct((EPH, 16), jnp.float32)],
        mesh=_mesh(),
        scratch_types=[pltpu.VMEM((G_CHUNK,), jnp.int32)] * 4
        + [pltpu.VMEM((G_CHUNK, 16), jnp.float32)] * 4
        + [pltpu.SemaphoreType.DMA] * 8,
        compiler_params=_sc_params,
    )


def _sc_gather(h16, src, dst):
    return _sc_gather_fn()(h16, src, dst)


def _scatter_core(dst2_hbm, z_hbm, out_hbm, idx2, agg_s, asem, load, src,
                  s_outer):
    c = lax.axis_index("c").astype(jnp.int32)
    s = lax.axis_index("s").astype(jnp.int32)
    w = _wid()
    pltpu.sync_copy(z_hbm,
                    agg_s.at[pl.ds(s * _i32(ROWS_PER_SUB), ROWS_PER_SUB)])
    plsc.subcore_barrier()

    def loads(t, p):
        base128 = w * _i32(s_outer * S_SUB) + t * _i32(S_SUB)
        pltpu.sync_copy(dst2_hbm.at[pl.ds(base128, S_SUB)], idx2[p])
        load(t, p)

    def adds(p):
        return [pltpu.async_copy(src(p, k), agg_s.at[idx2[p].at[_i32(k)]],
                                 asem[p], add=True)
                for k in range(S_SUB)]

    peel = s_outer % 2
    if peel:
        loads(_i32(0), 0)
        for d in adds(0):
            d.wait()

    def outer(jj, carry):
        t0 = _i32(peel) + jj * _i32(2)
        loads(t0, 0)
        ad0 = adds(0)
        loads(t0 + _i32(1), 1)
        ad1 = adds(1)
        for d in ad0 + ad1:
            d.wait()
        return carry

    lax.fori_loop(_i32(0), _i32((s_outer - peel) // 2), outer, _i32(0))
    plsc.subcore_barrier()
    pltpu.sync_copy(
        agg_s.at[pl.ds(s * _i32(ROWS_PER_SUB), ROWS_PER_SUB)],
        out_hbm.at[c, pl.ds(s * _i32(ROWS_PER_SUB), ROWS_PER_SUB)])


def _scatter_body(dst2_hbm, msg_hbm, z_hbm, out_hbm, idx_a, idx_b,
                  msg_a, msg_b, agg_s, as_a, as_b):
    w = _wid()
    msgs = (msg_a, msg_b)
    s_outer = S_PW // S_CHUNK

    def load(t, p):
        base = w * _i32(S_PW) + t * _i32(S_CHUNK)
        pltpu.sync_copy(msg_hbm.at[pl.ds(base, S_CHUNK)], msgs[p])

    def src(p, k):
        return msgs[p].at[pl.ds(_i32(k * 128), 128)]

    _scatter_core(dst2_hbm, z_hbm, out_hbm, (idx_a, idx_b), agg_s,
                  (as_a, as_b), load, src, s_outer)


def _scatter_const_body(dst2_hbm, m0_hbm, z_hbm, out_hbm, idx_a, idx_b,
                        m0_v, agg_s, as_a, as_b):
    pltpu.sync_copy(m0_hbm, m0_v)

    def load(t, p):
        pass

    def src(p, k):
        return m0_v

    _scatter_core(dst2_hbm, z_hbm, out_hbm, (idx_a, idx_b), agg_s,
                  (as_a, as_b), load, src, EP // NW // S_CHUNK)


_agg_out = jax.ShapeDtypeStruct((NC, NP, 16), jnp.float32)


@functools.lru_cache(maxsize=None)
def _sc_scatter_fn():
    return pl.kernel(
        _scatter_body, out_type=_agg_out, mesh=_mesh(),
        scratch_types=[pltpu.VMEM((S_SUB, 128), jnp.int32),
                       pltpu.VMEM((S_SUB, 128), jnp.int32),
                       pltpu.VMEM((S_CHUNK, 16), jnp.float32),
                       pltpu.VMEM((S_CHUNK, 16), jnp.float32),
                       pltpu.VMEM_SHARED((NP, 16), jnp.float32),
                       pltpu.SemaphoreType.DMA,
                       pltpu.SemaphoreType.DMA],
        compiler_params=_sc_params)


@functools.lru_cache(maxsize=None)
def _sc_scatter_const_fn():
    return pl.kernel(
        _scatter_const_body, out_type=_agg_out, mesh=_mesh(),
        scratch_types=[pltpu.VMEM((S_SUB, 128), jnp.int32),
                       pltpu.VMEM((S_SUB, 128), jnp.int32),
                       pltpu.VMEM((128, 16), jnp.float32),
                       pltpu.VMEM_SHARED((NP, 16), jnp.float32),
                       pltpu.SemaphoreType.DMA,
                       pltpu.SemaphoreType.DMA],
        compiler_params=_sc_params)


def _sc_scatter(dst2h, msgh, zrows):
    return _sc_scatter_fn()(dst2h, msgh, zrows)


def _sc_scatter_const(dst2, m0t, zrows):
    return _sc_scatter_const_fn()(dst2, m0t, zrows)


BMR = BM // 8


def _mlp_body(hs, hd, a1, b1, w2, w3, w4s, m1, m2, m3, m4, out):
    f32 = jnp.float32
    bf = jnp.bfloat16

    def dot(x, w):
        return jax.lax.dot(x.astype(bf), w, precision=None,
                           preferred_element_type=f32)

    x1 = dot(hs[...], a1[...]) + dot(hd[...], b1[...]) + m1[...][0:1]
    x1 = jnp.maximum(x1, 0.0)
    x2 = jnp.concatenate(
        [jnp.maximum(dot(x1[:, 128 * k:128 * (k + 1)], w2[...]) + m2[...][0:1],
                     0.0) for k in range(8)], axis=1)
    x3 = jnp.concatenate(
        [jnp.maximum(dot(x2[:, 128 * k:128 * (k + 1)], w3[...]) + m3[...][0:1],
                     0.0) for k in range(8)], axis=1)
    acc = m4[...][0:1] + jnp.zeros((BMR, 128), f32)
    for k in range(8):
        acc = acc + dot(x3[:, 128 * k:128 * (k + 1)],
                        w4s[...][128 * k:128 * (k + 1), :])
    out[...] = acc


def _full(shape):
    return pl.BlockSpec(shape, lambda i: (jnp.int32(0), jnp.int32(0)))


def _tc_mlp(hs, hd, a1, b1, w2, w3, w4s, m1, m2, m3, m4):
    grid = EPH // BM
    blk = pl.BlockSpec((BMR, 128), lambda i: (i, jnp.int32(0)))
    return pl.pallas_call(
        _mlp_body,
        grid=(grid,),
        in_specs=[blk, blk, _full((128, 1024)), _full((128, 1024)),
                  _full((128, 128)), _full((128, 128)), _full((1024, 128)),
                  _full((8, 1024)), _full((8, 128)), _full((8, 128)),
                  _full((8, 128))],
        out_specs=blk,
        out_shape=jax.ShapeDtypeStruct((EPH // 8, 128), jnp.float32),
        compiler_params=pltpu.CompilerParams(
            dimension_semantics=("arbitrary",)),
    )(hs, hd, a1, b1, w2, w3, w4s, m1, m2, m3, m4)


def _gru_body(a0, a1, ni, h, war, wnr, whr, waz, wnz, whz, wan, wnn,
              whn, br, bz, bn, bhr, bhz, bhn, fwp, fbp, hout, oout):
    agg = a0[...] + a1[...]
    nn = ni[...]
    hh = h[...]
    i_r = jnp.dot(agg, war[...]) + jnp.dot(nn, wnr[...]) + br[...][0:1]
    i_z = jnp.dot(agg, waz[...]) + jnp.dot(nn, wnz[...]) + bz[...][0:1]
    i_n = jnp.dot(agg, wan[...]) + jnp.dot(nn, wnn[...]) + bn[...][0:1]
    h_r = jnp.dot(hh, whr[...]) + bhr[...][0:1]
    h_z = jnp.dot(hh, whz[...]) + bhz[...][0:1]
    h_n = jnp.dot(hh, whn[...]) + bhn[...][0:1]
    r = jax.nn.sigmoid(i_r + h_r)
    z = jax.nn.sigmoid(i_z + h_z)
    n = jnp.tanh(i_n + r * h_n)
    hnew = (1.0 - z) * n + z * hh
    hout[...] = hnew
    oout[...] = jnp.dot(hnew, fwp[...]) + fbp[...][0:1]


BNR = BN // 8


def _tc_gru(a0, a1, ni, h, ws, bs, fwp, fbp):
    grid = NP // BN
    blk = pl.BlockSpec((BNR, 128), lambda i: (i, jnp.int32(0)))
    wspec = _full((128, 128))
    bspec = _full((8, 128))
    return pl.pallas_call(
        _gru_body,
        grid=(grid,),
        in_specs=[blk, blk, blk, blk] + [wspec] * 9 + [bspec] * 6
        + [wspec, bspec],
        out_specs=[blk, blk],
        out_shape=[jax.ShapeDtypeStruct((NP // 8, 128), jnp.float32),
                   jax.ShapeDtypeStruct((NP // 8, 128), jnp.float32)],
        compiler_params=pltpu.CompilerParams(
            dimension_semantics=("arbitrary",)),
    )(a0, a1, ni, h, *ws, *bs, fwp, fbp)


def kernel(node_inputs, src_ids, dst_ids, mw1, mb1, mw2, mb2, mw3, mb3, mw4,
           mb4, w_ih, w_hh, b_ih, b_hh, fw, fb):
    f32 = jnp.float32
    epad = EP - E
    src = jnp.concatenate([src_ids.astype(jnp.int32),
                           jnp.zeros((epad,), jnp.int32)])
    dst = jnp.concatenate([dst_ids.astype(jnp.int32),
                           jnp.full((epad,), TRASH, jnp.int32)])
    dst2 = dst.reshape(EP // 128, 128)

    ni128 = jnp.zeros((NP, 16), f32).at[:N, :NI].set(
        node_inputs.astype(f32)).reshape(NP // 8, 128)

    eye8 = jnp.eye(8, dtype=f32)

    def brow(vec16):
        return jnp.tile(jnp.tile(vec16, 8)[None, :], (8, 1))

    bf = jnp.bfloat16
    a1p = jnp.zeros((16, 128), f32).at[:NF, :MS].set(mw1[:, :NF].T)
    b1p = jnp.zeros((16, 128), f32).at[:NF, :MS].set(mw1[:, NF:].T)
    a1 = jnp.kron(eye8, a1p).astype(bf)
    b1 = jnp.kron(eye8, b1p).astype(bf)
    w2 = jnp.zeros((128, 128), f32).at[:MS, :MS].set(mw2.T).astype(bf)
    w3 = jnp.zeros((128, 128), f32).at[:MS, :MS].set(mw3.T).astype(bf)
    w4s = jnp.concatenate(
        [jnp.zeros((128, 128), f32).at[:MS, 16 * k:16 * k + EF].set(mw4.T)
         for k in range(8)], axis=0).astype(bf)
    m1 = jnp.tile(jnp.tile(jnp.zeros(128, f32).at[:MS].set(mb1), 8)[None, :],
                  (8, 1))
    m2 = jnp.tile(jnp.zeros(128, f32).at[:MS].set(mb2)[None, :], (8, 1))
    m3 = jnp.tile(jnp.zeros(128, f32).at[:MS].set(mb3)[None, :], (8, 1))
    m4 = brow(jnp.zeros(16, f32).at[:EF].set(mb4))

    def gw(mat, g, in_lo, in_hi, in_n):
        out = jnp.zeros((16, 16), f32)
        out = out.at[:in_n, :NF].set(mat[NF * g:NF * (g + 1), in_lo:in_hi].T)
        return jnp.kron(eye8, out)

    ws = []
    for g in range(3):
        ws.append(gw(w_ih, g, 0, EF, EF))
        ws.append(gw(w_ih, g, EF, EF + NI, NI))
        ws.append(gw(w_hh, g, 0, NF, NF))

    def gb(vec, g):
        return brow(jnp.zeros(16, f32).at[:NF].set(vec[NF * g:NF * (g + 1)]))

    bs = [gb(b_ih, 0), gb(b_ih, 1), gb(b_ih, 2),
          gb(b_hh, 0), gb(b_hh, 1), gb(b_hh, 2)]

    fwp = jnp.kron(eye8, jnp.zeros((16, 16), f32).at[:NF, :NO].set(fw.T))
    fbp = brow(jnp.zeros(16, f32).at[:NO].set(fb))

    h1_ = jnp.maximum(mb1[None, :], 0.0)
    h2_ = jnp.maximum(jnp.dot(h1_, mw2.T) + mb2[None, :], 0.0)
    h3_ = jnp.maximum(jnp.dot(h2_, mw3.T) + mb3[None, :], 0.0)
    m0 = jnp.dot(h3_, mw4.T) + mb4[None, :]
    m0t = jnp.zeros((128, 16), f32).at[:, :EF].set(m0)

    zrows = jnp.zeros((ROWS_PER_SUB, 16), f32)
    h128 = jnp.zeros((NP // 8, 128), f32)
    zagg = jnp.zeros((NP // 8, 128), f32)

    outs = []
    agg0 = _sc_scatter_const(dst2, m0t, zrows).reshape(NC, NP // 8, 128)
    aggs = (agg0[0], agg0[1])
    for it in range(ITERS):
        h128, o = _tc_gru(*aggs, ni128, h128, ws, bs, fwp, fbp)
        outs.append(o.reshape(NP, 16)[:N, :NO])
        if it < ITERS - 1:
            h16sc = h128.reshape(NP, 16)
            hs, hd = _sc_gather(h16sc, src, dst)
            msg = _tc_mlp(hs.reshape(EPH // 8, 128),
                          hd.reshape(EPH // 8, 128),
                          a1, b1, w2, w3, w4s, m1, m2, m3, m4)
            ap = _sc_scatter(dst2, msg.reshape(EPH, 16),
                             zrows).reshape(NC, NP // 8, 128)
            aggs = (ap[0], ap[1])
    return jnp.stack(outs, axis=0)


# speedup vs baseline: 1.2754x; 1.0184x over previous
import functools

import jax
import jax.numpy as jnp
from jax import lax
from jax.experimental import pallas as pl
from jax.experimental.pallas import tpu as pltpu, tpu_sc as plsc

N = 50000
E = 800000
NF = 10
NI = 9
EF = 11
NO = 9
MS = 96
ITERS = 3

NC = 2
NS = 16
NW = NC * NS

EP = 819200
EPH = EP
G_CHUNK = 1600
GA = 22
GB = 10
S_CHUNK = 1024
S_PW = EPH // NW
S_SUB = S_CHUNK // 128

NP = 50176
ROWS_PER_SUB = NP // NS
TRASH = N

BM = 16384
BN = 6272

@functools.lru_cache(maxsize=None)
def _mesh():
    return plsc.VectorSubcoreMesh(core_axis_name="c", subcore_axis_name="s",
                                  num_cores=NC, num_subcores=NS)


_sc_params = pltpu.CompilerParams(use_tc_tiling_on_sc=False)


def _i32(x):
    return jnp.asarray(x, jnp.int32)


def _wid():
    return (lax.axis_index("s").astype(jnp.int32) * _i32(NC)
            + lax.axis_index("c").astype(jnp.int32))


def _gather_body(h_hbm, src_hbm, dst_hbm, hs_out, hd_out,
                 is_a, is_b, id_a, id_b, rs_a, rs_b, rd_a, rd_b,
                 gsa, gsb, gda, gdb, wsa, wsb, wda, wdb):
    c = lax.axis_index("c").astype(jnp.int32)
    s = lax.axis_index("s").astype(jnp.int32)
    sidx = (is_a, is_b)
    didx = (id_a, id_b)
    srow = (rs_a, rs_b)
    drow = (rd_a, rd_b)
    gs = (gsa, gsb)
    gd = (gda, gdb)
    ws = (wsa, wsb)
    wd = (wda, wdb)

    chunk0 = jnp.where(c == 0, s * _i32(GA),
                       _i32(NS * GA) + s * _i32(GB))
    my_t = jnp.where(c == 0, _i32(GA), _i32(GB))

    def base(j):
        return (chunk0 + j) * _i32(G_CHUNK)

    def start(j, p):
        b = base(j)
        pltpu.sync_copy(src_hbm.at[pl.ds(b, G_CHUNK)], sidx[p])
        pltpu.async_copy(h_hbm.at[sidx[p]], srow[p], gs[p])
        pltpu.sync_copy(dst_hbm.at[pl.ds(b, G_CHUNK)], didx[p])
        pltpu.async_copy(h_hbm.at[didx[p]], drow[p], gd[p])

    def finish(j, p):
        b = pl.ds(base(j), G_CHUNK)
        pltpu.make_async_copy(h_hbm.at[sidx[p]], srow[p], gs[p]).wait()
        pltpu.async_copy(srow[p], hs_out.at[b], ws[p])
        pltpu.make_async_copy(h_hbm.at[didx[p]], drow[p], gd[p]).wait()
        pltpu.async_copy(drow[p], hd_out.at[b], wd[p])

    def drain_wb(p):
        b0 = pl.ds(_i32(0), G_CHUNK)
        pltpu.make_async_copy(srow[p], hs_out.at[b0], ws[p]).wait()
        pltpu.make_async_copy(drow[p], hd_out.at[b0], wd[p]).wait()

    start(_i32(0), 0)
    start(_i32(1), 1)

    def outer(jj, carry):
        j0 = jj * _i32(2)
        finish(j0, 0)
        finish(j0 + _i32(1), 1)
        drain_wb(0)
        start(j0 + _i32(2), 0)
        drain_wb(1)
        start(j0 + _i32(3), 1)
        return carry

    lax.fori_loop(_i32(0), my_t // _i32(2) - _i32(1), outer, _i32(0))
    finish(my_t - _i32(2), 0)
    finish(my_t - _i32(1), 1)
    drain_wb(0)
    drain_wb(1)


@functools.lru_cache(maxsize=None)
def _sc_gather_fn():
    return pl.kernel(
        _gather_body,
        out_type=[jax.ShapeDtypeStruct((EPH, 16), jnp.float32),
                  jax.ShapeDtypeStruct((EPH, 16), jnp.float32)],
        mesh=_mesh(),
        scratch_types=[pltpu.VMEM((G_CHUNK,), jnp.int32)] * 4
        + [pltpu.VMEM((G_CHUNK, 16), jnp.float32)] * 4
        + [pltpu.SemaphoreType.DMA] * 8,
        compiler_params=_sc_params,
    )


def _sc_gather(h16, src, dst):
    return _sc_gather_fn()(h16, src, dst)


def _scatter_core(dst2_hbm, z_hbm, out_hbm, idx2, agg_s, asem, load, src,
                  s_outer):
    c = lax.axis_index("c").astype(jnp.int32)
    s = lax.axis_index("s").astype(jnp.int32)
    w = _wid()
    pltpu.sync_copy(z_hbm,
                    agg_s.at[pl.ds(s * _i32(ROWS_PER_SUB), ROWS_PER_SUB)])
    plsc.subcore_barrier()

    def loads(t, p):
        base128 = w * _i32(s_outer * S_SUB) + t * _i32(S_SUB)
        pltpu.sync_copy(dst2_hbm.at[pl.ds(base128, S_SUB)], idx2[p])
        load(t, p)

    def adds(p):
        return [pltpu.async_copy(src(p, k), agg_s.at[idx2[p].at[_i32(k)]],
                                 asem[p], add=True)
                for k in range(S_SUB)]

    peel = s_outer % 2
    if peel:
        loads(_i32(0), 0)
        for d in adds(0):
            d.wait()

    def outer(jj, carry):
        t0 = _i32(peel) + jj * _i32(2)
        loads(t0, 0)
        ad0 = adds(0)
        loads(t0 + _i32(1), 1)
        ad1 = adds(1)
        for d in ad0 + ad1:
            d.wait()
        return carry

    lax.fori_loop(_i32(0), _i32((s_outer - peel) // 2), outer, _i32(0))
    plsc.subcore_barrier()
    pltpu.sync_copy(
        agg_s.at[pl.ds(s * _i32(ROWS_PER_SUB), ROWS_PER_SUB)],
        out_hbm.at[c, pl.ds(s * _i32(ROWS_PER_SUB), ROWS_PER_SUB)])


def _scatter_body(dst2_hbm, msg_hbm, z_hbm, out_hbm, idx_a, idx_b,
                  msg_a, msg_b, agg_s, as_a, as_b):
    w = _wid()
    msgs = (msg_a, msg_b)
    s_outer = S_PW // S_CHUNK

    def load(t, p):
        base = w * _i32(S_PW) + t * _i32(S_CHUNK)
        pltpu.sync_copy(msg_hbm.at[pl.ds(base, S_CHUNK)], msgs[p])

    def src(p, k):
        return msgs[p].at[pl.ds(_i32(k * 128), 128)]

    _scatter_core(dst2_hbm, z_hbm, out_hbm, (idx_a, idx_b), agg_s,
                  (as_a, as_b), load, src, s_outer)


def _scatter_const_body(dst2_hbm, m0_hbm, z_hbm, out_hbm, idx_a, idx_b,
                        m0_v, agg_s, as_a, as_b):
    pltpu.sync_copy(m0_hbm, m0_v)

    def load(t, p):
        pass

    def src(p, k):
        return m0_v

    _scatter_core(dst2_hbm, z_hbm, out_hbm, (idx_a, idx_b), agg_s,
                  (as_a, as_b), load, src, EP // NW // S_CHUNK)


_agg_out = jax.ShapeDtypeStruct((NC, NP, 16), jnp.float32)


@functools.lru_cache(maxsize=None)
def _sc_scatter_fn():
    return pl.kernel(
        _scatter_body, out_type=_agg_out, mesh=_mesh(),
        scratch_types=[pltpu.VMEM((S_SUB, 128), jnp.int32),
                       pltpu.VMEM((S_SUB, 128), jnp.int32),
                       pltpu.VMEM((S_CHUNK, 16), jnp.float32),
                       pltpu.VMEM((S_CHUNK, 16), jnp.float32),
                       pltpu.VMEM_SHARED((NP, 16), jnp.float32),
                       pltpu.SemaphoreType.DMA,
                       pltpu.SemaphoreType.DMA],
        compiler_params=_sc_params)


@functools.lru_cache(maxsize=None)
def _sc_scatter_const_fn():
    return pl.kernel(
        _scatter_const_body, out_type=_agg_out, mesh=_mesh(),
        scratch_types=[pltpu.VMEM((S_SUB, 128), jnp.int32),
                       pltpu.VMEM((S_SUB, 128), jnp.int32),
                       pltpu.VMEM((128, 16), jnp.float32),
                       pltpu.VMEM_SHARED((NP, 16), jnp.float32),
                       pltpu.SemaphoreType.DMA,
                       pltpu.SemaphoreType.DMA],
        compiler_params=_sc_params)


def _sc_scatter(dst2h, msgh, zrows):
    return _sc_scatter_fn()(dst2h, msgh, zrows)


def _sc_scatter_const(dst2, m0t, zrows):
    return _sc_scatter_const_fn()(dst2, m0t, zrows)


BMR = BM // 8


def _mlp_body(hs, hd, a1, b1, w2, w3, w4s, m1, m2, m3, m4, out):
    f32 = jnp.float32
    bf = jnp.bfloat16

    def dot(x, w):
        return jax.lax.dot(x.astype(bf), w, precision=None,
                           preferred_element_type=f32)

    x1 = dot(hs[...], a1[...]) + dot(hd[...], b1[...]) + m1[...][0:1]
    x1 = jnp.maximum(x1, 0.0)
    x2 = jnp.concatenate(
        [jnp.maximum(dot(x1[:, 128 * k:128 * (k + 1)], w2[...]) + m2[...][0:1],
                     0.0) for k in range(8)], axis=1)
    x3 = jnp.concatenate(
        [jnp.maximum(dot(x2[:, 128 * k:128 * (k + 1)], w3[...]) + m3[...][0:1],
                     0.0) for k in range(8)], axis=1)
    acc = m4[...][0:1] + jnp.zeros((BMR, 128), f32)
    for k in range(8):
        acc = acc + dot(x3[:, 128 * k:128 * (k + 1)],
                        w4s[...][128 * k:128 * (k + 1), :])
    out[...] = acc


def _full(shape):
    return pl.BlockSpec(shape, lambda i: (jnp.int32(0), jnp.int32(0)))


def _tc_mlp(hs, hd, a1, b1, w2, w3, w4s, m1, m2, m3, m4):
    grid = EPH // BM
    blk = pl.BlockSpec((BMR, 128), lambda i: (i, jnp.int32(0)))
    return pl.pallas_call(
        _mlp_body,
        grid=(grid,),
        in_specs=[blk, blk, _full((128, 1024)), _full((128, 1024)),
                  _full((128, 128)), _full((128, 128)), _full((1024, 128)),
                  _full((8, 1024)), _full((8, 128)), _full((8, 128)),
                  _full((8, 128))],
        out_specs=blk,
        out_shape=jax.ShapeDtypeStruct((EPH // 8, 128), jnp.float32),
        compiler_params=pltpu.CompilerParams(
            dimension_semantics=("arbitrary",)),
    )(hs, hd, a1, b1, w2, w3, w4s, m1, m2, m3, m4)


def _gru_body(a0, a1, ni, h, war, wnr, whr, waz, wnz, whz, wan, wnn,
              whn, br, bz, bn, bhr, bhz, bhn, fwp, fbp, hout, oout):
    agg = a0[...] + a1[...]
    nn = ni[...]
    hh = h[...]
    i_r = jnp.dot(agg, war[...]) + jnp.dot(nn, wnr[...]) + br[...][0:1]
    i_z = jnp.dot(agg, waz[...]) + jnp.dot(nn, wnz[...]) + bz[...][0:1]
    i_n = jnp.dot(agg, wan[...]) + jnp.dot(nn, wnn[...]) + bn[...][0:1]
    h_r = jnp.dot(hh, whr[...]) + bhr[...][0:1]
    h_z = jnp.dot(hh, whz[...]) + bhz[...][0:1]
    h_n = jnp.dot(hh, whn[...]) + bhn[...][0:1]
    r = jax.nn.sigmoid(i_r + h_r)
    z = jax.nn.sigmoid(i_z + h_z)
    n = jnp.tanh(i_n + r * h_n)
    hnew = (1.0 - z) * n + z * hh
    hout[...] = hnew
    oout[...] = jnp.dot(hnew, fwp[...]) + fbp[...][0:1]


BNR = BN // 8


def _tc_gru(a0, a1, ni, h, ws, bs, fwp, fbp):
    grid = NP // BN
    blk = pl.BlockSpec((BNR, 128), lambda i: (i, jnp.int32(0)))
    wspec = _full((128, 128))
    bspec = _full((8, 128))
    return pl.pallas_call(
        _gru_body,
        grid=(grid,),
        in_specs=[blk, blk, blk, blk] + [wspec] * 9 + [bspec] * 6
        + [wspec, bspec],
        out_specs=[blk, blk],
        out_shape=[jax.ShapeDtypeStruct((NP // 8, 128), jnp.float32),
                   jax.ShapeDtypeStruct((NP // 8, 128), jnp.float32)],
        compiler_params=pltpu.CompilerParams(
            dimension_semantics=("arbitrary",)),
    )(a0, a1, ni, h, *ws, *bs, fwp, fbp)


def kernel(node_inputs, src_ids, dst_ids, mw1, mb1, mw2, mb2, mw3, mb3, mw4,
           mb4, w_ih, w_hh, b_ih, b_hh, fw, fb):
    f32 = jnp.float32
    epad = EP - E
    src = jnp.concatenate([src_ids.astype(jnp.int32),
                           jnp.zeros((epad,), jnp.int32)])
    dst = jnp.concatenate([dst_ids.astype(jnp.int32),
                           jnp.full((epad,), TRASH, jnp.int32)])
    dst2 = dst.reshape(EP // 128, 128)

    ni128 = jnp.zeros((NP, 16), f32).at[:N, :NI].set(
        node_inputs.astype(f32)).reshape(NP // 8, 128)

    eye8 = jnp.eye(8, dtype=f32)

    def brow(vec16):
        return jnp.tile(jnp.tile(vec16, 8)[None, :], (8, 1))

    bf = jnp.bfloat16
    a1p = jnp.zeros((16, 128), f32).at[:NF, :MS].set(mw1[:, :NF].T)
    b1p = jnp.zeros((16, 128), f32).at[:NF, :MS].set(mw1[:, NF:].T)
    a1 = jnp.kron(eye8, a1p).astype(bf)
    b1 = jnp.kron(eye8, b1p).astype(bf)
    w2 = jnp.zeros((128, 128), f32).at[:MS, :MS].set(mw2.T).astype(bf)
    w3 = jnp.zeros((128, 128), f32).at[:MS, :MS].set(mw3.T).astype(bf)
    w4s = jnp.concatenate(
        [jnp.zeros((128, 128), f32).at[:MS, 16 * k:16 * k + EF].set(mw4.T)
         for k in range(8)], axis=0).astype(bf)
    m1 = jnp.tile(jnp.tile(jnp.zeros(128, f32).at[:MS].set(mb1), 8)[None, :],
                  (8, 1))
    m2 = jnp.tile(jnp.zeros(128, f32).at[:MS].set(mb2)[None, :], (8, 1))
    m3 = jnp.tile(jnp.zeros(128, f32).at[:MS].set(mb3)[None, :], (8, 1))
    m4 = brow(jnp.zeros(16, f32).at[:EF].set(mb4))

    def gw(mat, g, in_lo, in_hi, in_n):
        out = jnp.zeros((16, 16), f32)
        out = out.at[:in_n, :NF].set(mat[NF * g:NF * (g + 1), in_lo:in_hi].T)
        return jnp.kron(eye8, out)

    ws = []
    for g in range(3):
        ws.append(gw(w_ih, g, 0, EF, EF))
        ws.append(gw(w_ih, g, EF, EF + NI, NI))
        ws.append(gw(w_hh, g, 0, NF, NF))

    def gb(vec, g):
        return brow(jnp.zeros(16, f32).at[:NF].set(vec[NF * g:NF * (g + 1)]))

    bs = [gb(b_ih, 0), gb(b_ih, 1), gb(b_ih, 2),
          gb(b_hh, 0), gb(b_hh, 1), gb(b_hh, 2)]

    fwp = jnp.kron(eye8, jnp.zeros((16, 16), f32).at[:NF, :NO].set(fw.T))
    fbp = brow(jnp.zeros(16, f32).at[:NO].set(fb))

    h1_ = jnp.maximum(mb1[None, :], 0.0)
    h2_ = jnp.maximum(jnp.dot(h1_, mw2.T) + mb2[None, :], 0.0)
    h3_ = jnp.maximum(jnp.dot(h2_, mw3.T) + mb3[None, :], 0.0)
    m0 = jnp.dot(h3_, mw4.T) + mb4[None, :]
    m0t = jnp.zeros((128, 16), f32).at[:, :EF].set(m0)

    zrows = jnp.zeros((ROWS_PER_SUB, 16), f32)
    h128 = jnp.zeros((NP // 8, 128), f32)
    zagg = jnp.zeros((NP // 8, 128), f32)

    outs = []
    agg0 = _sc_scatter_const(dst2, m0t, zrows).reshape(NC, NP // 8, 128)
    aggs = (agg0[0], agg0[1])
    for it in range(ITERS):
        h128, o = _tc_gru(*aggs, ni128, h128, ws, bs, fwp, fbp)
        outs.append(o.reshape(NP, 16)[:N, :NO])
        if it < ITERS - 1:
            h16sc = h128.reshape(NP, 16)
            hs, hd = _sc_gather(h16sc, src, dst)
            msg = _tc_mlp(hs.reshape(EPH // 8, 128),
                          hd.reshape(EPH // 8, 128),
                          a1, b1, w2, w3, w4s, m1, m2, m3, m4)
            ap = _sc_scatter(dst2, msg.reshape(EPH, 16),
                             zrows).reshape(NC, NP // 8, 128)
            aggs = (ap[0], ap[1])
    return jnp.stack(outs, axis=0)


# speedup vs baseline: 1.2813x; 1.0047x over previous
import functools

import jax
import jax.numpy as jnp
from jax import lax
from jax.experimental import pallas as pl
from jax.experimental.pallas import tpu as pltpu, tpu_sc as plsc

N = 50000
E = 800000
NF = 10
NI = 9
EF = 11
NO = 9
MS = 96
ITERS = 3

NC = 2
NS = 16
NW = NC * NS

EP = 819200
EPH = EP
G_CHUNK = 1600
GA = 26
GB = 6
S_CHUNK = 1024
S_PW = EPH // NW
S_SUB = S_CHUNK // 128

NP = 50176
ROWS_PER_SUB = NP // NS
TRASH = N

BM = 16384
BN = 6272

@functools.lru_cache(maxsize=None)
def _mesh():
    return plsc.VectorSubcoreMesh(core_axis_name="c", subcore_axis_name="s",
                                  num_cores=NC, num_subcores=NS)


_sc_params = pltpu.CompilerParams(use_tc_tiling_on_sc=False)


def _i32(x):
    return jnp.asarray(x, jnp.int32)


def _wid():
    return (lax.axis_index("s").astype(jnp.int32) * _i32(NC)
            + lax.axis_index("c").astype(jnp.int32))


def _gather_body(h_hbm, src_hbm, dst_hbm, hs_out, hd_out,
                 is_a, is_b, id_a, id_b, rs_a, rs_b, rd_a, rd_b,
                 gsa, gsb, gda, gdb, wsa, wsb, wda, wdb):
    c = lax.axis_index("c").astype(jnp.int32)
    s = lax.axis_index("s").astype(jnp.int32)
    sidx = (is_a, is_b)
    didx = (id_a, id_b)
    srow = (rs_a, rs_b)
    drow = (rd_a, rd_b)
    gs = (gsa, gsb)
    gd = (gda, gdb)
    ws = (wsa, wsb)
    wd = (wda, wdb)

    chunk0 = jnp.where(c == 0, s * _i32(GA),
                       _i32(NS * GA) + s * _i32(GB))
    my_t = jnp.where(c == 0, _i32(GA), _i32(GB))

    def base(j):
        return (chunk0 + j) * _i32(G_CHUNK)

    def start(j, p):
        b = base(j)
        pltpu.sync_copy(src_hbm.at[pl.ds(b, G_CHUNK)], sidx[p])
        pltpu.async_copy(h_hbm.at[sidx[p]], srow[p], gs[p])
        pltpu.sync_copy(dst_hbm.at[pl.ds(b, G_CHUNK)], didx[p])
        pltpu.async_copy(h_hbm.at[didx[p]], drow[p], gd[p])

    def finish(j, p):
        b = pl.ds(base(j), G_CHUNK)
        pltpu.make_async_copy(h_hbm.at[sidx[p]], srow[p], gs[p]).wait()
        pltpu.async_copy(srow[p], hs_out.at[b], ws[p])
        pltpu.make_async_copy(h_hbm.at[didx[p]], drow[p], gd[p]).wait()
        pltpu.async_copy(drow[p], hd_out.at[b], wd[p])

    def drain_wb(p):
        b0 = pl.ds(_i32(0), G_CHUNK)
        pltpu.make_async_copy(srow[p], hs_out.at[b0], ws[p]).wait()
        pltpu.make_async_copy(drow[p], hd_out.at[b0], wd[p]).wait()

    start(_i32(0), 0)
    start(_i32(1), 1)

    def outer(jj, carry):
        j0 = jj * _i32(2)
        finish(j0, 0)
        finish(j0 + _i32(1), 1)
        drain_wb(0)
        start(j0 + _i32(2), 0)
        drain_wb(1)
        start(j0 + _i32(3), 1)
        return carry

    lax.fori_loop(_i32(0), my_t // _i32(2) - _i32(1), outer, _i32(0))
    finish(my_t - _i32(2), 0)
    finish(my_t - _i32(1), 1)
    drain_wb(0)
    drain_wb(1)


@functools.lru_cache(maxsize=None)
def _sc_gather_fn():
    return pl.kernel(
        _gather_body,
        out_type=[jax.ShapeDtypeStruct((EPH, 16), jnp.float32),
                  jax.ShapeDtypeStruct((EPH, 16), jnp.float32)],
        mesh=_mesh(),
        scratch_types=[pltpu.VMEM((G_CHUNK,), jnp.int32)] * 4
        + [pltpu.VMEM((G_CHUNK, 16), jnp.float32)] * 4
        + [pltpu.SemaphoreType.DMA] * 8,
        compiler_params=_sc_params,
    )


def _sc_gather(h16, src, dst):
    return _sc_gather_fn()(h16, src, dst)


def _scatter_core(dst2_hbm, z_hbm, out_hbm, idx2, agg_s, asem, load, src,
                  s_outer):
    c = lax.axis_index("c").astype(jnp.int32)
    s = lax.axis_index("s").astype(jnp.int32)
    w = _wid()
    pltpu.sync_copy(z_hbm,
                    agg_s.at[pl.ds(s * _i32(ROWS_PER_SUB), ROWS_PER_SUB)])
    plsc.subcore_barrier()

    def loads(t, p):
        base128 = w * _i32(s_outer * S_SUB) + t * _i32(S_SUB)
        pltpu.sync_copy(dst2_hbm.at[pl.ds(base128, S_SUB)], idx2[p])
        load(t, p)

    def adds(p):
        return [pltpu.async_copy(src(p, k), agg_s.at[idx2[p].at[_i32(k)]],
                                 asem[p], add=True)
                for k in range(S_SUB)]

    peel = s_outer % 2
    if peel:
        loads(_i32(0), 0)
        for d in adds(0):
            d.wait()

    def outer(jj, carry):
        t0 = _i32(peel) + jj * _i32(2)
        loads(t0, 0)
        ad0 = adds(0)
        loads(t0 + _i32(1), 1)
        ad1 = adds(1)
        for d in ad0 + ad1:
            d.wait()
        return carry

    lax.fori_loop(_i32(0), _i32((s_outer - peel) // 2), outer, _i32(0))
    plsc.subcore_barrier()
    pltpu.sync_copy(
        agg_s.at[pl.ds(s * _i32(ROWS_PER_SUB), ROWS_PER_SUB)],
        out_hbm.at[c, pl.ds(s * _i32(ROWS_PER_SUB), ROWS_PER_SUB)])


def _scatter_body(dst2_hbm, msg_hbm, z_hbm, out_hbm, idx_a, idx_b,
                  msg_a, msg_b, agg_s, as_a, as_b):
    w = _wid()
    msgs = (msg_a, msg_b)
    s_outer = S_PW // S_CHUNK

    def load(t, p):
        base = w * _i32(S_PW) + t * _i32(S_CHUNK)
        pltpu.sync_copy(msg_hbm.at[pl.ds(base, S_CHUNK)], msgs[p])

    def src(p, k):
        return msgs[p].at[pl.ds(_i32(k * 128), 128)]

    _scatter_core(dst2_hbm, z_hbm, out_hbm, (idx_a, idx_b), agg_s,
                  (as_a, as_b), load, src, s_outer)


def _scatter_const_body(dst2_hbm, m0_hbm, z_hbm, out_hbm, idx_a, idx_b,
                        m0_v, agg_s, as_a, as_b):
    pltpu.sync_copy(m0_hbm, m0_v)

    def load(t, p):
        pass

    def src(p, k):
        return m0_v

    _scatter_core(dst2_hbm, z_hbm, out_hbm, (idx_a, idx_b), agg_s,
                  (as_a, as_b), load, src, EP // NW // S_CHUNK)


_agg_out = jax.ShapeDtypeStruct((NC, NP, 16), jnp.float32)


@functools.lru_cache(maxsize=None)
def _sc_scatter_fn():
    return pl.kernel(
        _scatter_body, out_type=_agg_out, mesh=_mesh(),
        scratch_types=[pltpu.VMEM((S_SUB, 128), jnp.int32),
                       pltpu.VMEM((S_SUB, 128), jnp.int32),
                       pltpu.VMEM((S_CHUNK, 16), jnp.float32),
                       pltpu.VMEM((S_CHUNK, 16), jnp.float32),
                       pltpu.VMEM_SHARED((NP, 16), jnp.float32),
                       pltpu.SemaphoreType.DMA,
                       pltpu.SemaphoreType.DMA],
        compiler_params=_sc_params)


@functools.lru_cache(maxsize=None)
def _sc_scatter_const_fn():
    return pl.kernel(
        _scatter_const_body, out_type=_agg_out, mesh=_mesh(),
        scratch_types=[pltpu.VMEM((S_SUB, 128), jnp.int32),
                       pltpu.VMEM((S_SUB, 128), jnp.int32),
                       pltpu.VMEM((128, 16), jnp.float32),
                       pltpu.VMEM_SHARED((NP, 16), jnp.float32),
                       pltpu.SemaphoreType.DMA,
                       pltpu.SemaphoreType.DMA],
        compiler_params=_sc_params)


def _sc_scatter(dst2h, msgh, zrows):
    return _sc_scatter_fn()(dst2h, msgh, zrows)


def _sc_scatter_const(dst2, m0t, zrows):
    return _sc_scatter_const_fn()(dst2, m0t, zrows)


BMR = BM // 8


def _mlp_body(hs, hd, a1, b1, w2, w3, w4s, m1, m2, m3, m4, out):
    f32 = jnp.float32
    bf = jnp.bfloat16

    def dot(x, w):
        return jax.lax.dot(x.astype(bf), w, precision=None,
                           preferred_element_type=f32)

    x1 = dot(hs[...], a1[...]) + dot(hd[...], b1[...]) + m1[...][0:1]
    x1 = jnp.maximum(x1, 0.0)
    x2 = jnp.concatenate(
        [jnp.maximum(dot(x1[:, 128 * k:128 * (k + 1)], w2[...]) + m2[...][0:1],
                     0.0) for k in range(8)], axis=1)
    x3 = jnp.concatenate(
        [jnp.maximum(dot(x2[:, 128 * k:128 * (k + 1)], w3[...]) + m3[...][0:1],
                     0.0) for k in range(8)], axis=1)
    acc = m4[...][0:1] + jnp.zeros((BMR, 128), f32)
    for k in range(8):
        acc = acc + dot(x3[:, 128 * k:128 * (k + 1)],
                        w4s[...][128 * k:128 * (k + 1), :])
    out[...] = acc


def _full(shape):
    return pl.BlockSpec(shape, lambda i: (jnp.int32(0), jnp.int32(0)))


def _tc_mlp(hs, hd, a1, b1, w2, w3, w4s, m1, m2, m3, m4):
    grid = EPH // BM
    blk = pl.BlockSpec((BMR, 128), lambda i: (i, jnp.int32(0)))
    return pl.pallas_call(
        _mlp_body,
        grid=(grid,),
        in_specs=[blk, blk, _full((128, 1024)), _full((128, 1024)),
                  _full((128, 128)), _full((128, 128)), _full((1024, 128)),
                  _full((8, 1024)), _full((8, 128)), _full((8, 128)),
                  _full((8, 128))],
        out_specs=blk,
        out_shape=jax.ShapeDtypeStruct((EPH // 8, 128), jnp.float32),
        compiler_params=pltpu.CompilerParams(
            dimension_semantics=("arbitrary",)),
    )(hs, hd, a1, b1, w2, w3, w4s, m1, m2, m3, m4)


def _gru_body(a0, a1, ni, h, war, wnr, whr, waz, wnz, whz, wan, wnn,
              whn, br, bz, bn, bhr, bhz, bhn, fwp, fbp, hout, oout):
    agg = a0[...] + a1[...]
    nn = ni[...]
    hh = h[...]
    i_r = jnp.dot(agg, war[...]) + jnp.dot(nn, wnr[...]) + br[...][0:1]
    i_z = jnp.dot(agg, waz[...]) + jnp.dot(nn, wnz[...]) + bz[...][0:1]
    i_n = jnp.dot(agg, wan[...]) + jnp.dot(nn, wnn[...]) + bn[...][0:1]
    h_r = jnp.dot(hh, whr[...]) + bhr[...][0:1]
    h_z = jnp.dot(hh, whz[...]) + bhz[...][0:1]
    h_n = jnp.dot(hh, whn[...]) + bhn[...][0:1]
    r = jax.nn.sigmoid(i_r + h_r)
    z = jax.nn.sigmoid(i_z + h_z)
    n = jnp.tanh(i_n + r * h_n)
    hnew = (1.0 - z) * n + z * hh
    hout[...] = hnew
    oout[...] = jnp.dot(hnew, fwp[...]) + fbp[...][0:1]


BNR = BN // 8


def _tc_gru(a0, a1, ni, h, ws, bs, fwp, fbp):
    grid = NP // BN
    blk = pl.BlockSpec((BNR, 128), lambda i: (i, jnp.int32(0)))
    wspec = _full((128, 128))
    bspec = _full((8, 128))
    return pl.pallas_call(
        _gru_body,
        grid=(grid,),
        in_specs=[blk, blk, blk, blk] + [wspec] * 9 + [bspec] * 6
        + [wspec, bspec],
        out_specs=[blk, blk],
        out_shape=[jax.ShapeDtypeStruct((NP // 8, 128), jnp.float32),
                   jax.ShapeDtypeStruct((NP // 8, 128), jnp.float32)],
        compiler_params=pltpu.CompilerParams(
            dimension_semantics=("arbitrary",)),
    )(a0, a1, ni, h, *ws, *bs, fwp, fbp)


def kernel(node_inputs, src_ids, dst_ids, mw1, mb1, mw2, mb2, mw3, mb3, mw4,
           mb4, w_ih, w_hh, b_ih, b_hh, fw, fb):
    f32 = jnp.float32
    epad = EP - E
    src = jnp.concatenate([src_ids.astype(jnp.int32),
                           jnp.zeros((epad,), jnp.int32)])
    dst = jnp.concatenate([dst_ids.astype(jnp.int32),
                           jnp.full((epad,), TRASH, jnp.int32)])
    dst2 = dst.reshape(EP // 128, 128)

    ni128 = jnp.zeros((NP, 16), f32).at[:N, :NI].set(
        node_inputs.astype(f32)).reshape(NP // 8, 128)

    eye8 = jnp.eye(8, dtype=f32)

    def brow(vec16):
        return jnp.tile(jnp.tile(vec16, 8)[None, :], (8, 1))

    bf = jnp.bfloat16
    a1p = jnp.zeros((16, 128), f32).at[:NF, :MS].set(mw1[:, :NF].T)
    b1p = jnp.zeros((16, 128), f32).at[:NF, :MS].set(mw1[:, NF:].T)
    a1 = jnp.kron(eye8, a1p).astype(bf)
    b1 = jnp.kron(eye8, b1p).astype(bf)
    w2 = jnp.zeros((128, 128), f32).at[:MS, :MS].set(mw2.T).astype(bf)
    w3 = jnp.zeros((128, 128), f32).at[:MS, :MS].set(mw3.T).astype(bf)
    w4s = jnp.concatenate(
        [jnp.zeros((128, 128), f32).at[:MS, 16 * k:16 * k + EF].set(mw4.T)
         for k in range(8)], axis=0).astype(bf)
    m1 = jnp.tile(jnp.tile(jnp.zeros(128, f32).at[:MS].set(mb1), 8)[None, :],
                  (8, 1))
    m2 = jnp.tile(jnp.zeros(128, f32).at[:MS].set(mb2)[None, :], (8, 1))
    m3 = jnp.tile(jnp.zeros(128, f32).at[:MS].set(mb3)[None, :], (8, 1))
    m4 = brow(jnp.zeros(16, f32).at[:EF].set(mb4))

    def gw(mat, g, in_lo, in_hi, in_n):
        out = jnp.zeros((16, 16), f32)
        out = out.at[:in_n, :NF].set(mat[NF * g:NF * (g + 1), in_lo:in_hi].T)
        return jnp.kron(eye8, out)

    ws = []
    for g in range(3):
        ws.append(gw(w_ih, g, 0, EF, EF))
        ws.append(gw(w_ih, g, EF, EF + NI, NI))
        ws.append(gw(w_hh, g, 0, NF, NF))

    def gb(vec, g):
        return brow(jnp.zeros(16, f32).at[:NF].set(vec[NF * g:NF * (g + 1)]))

    bs = [gb(b_ih, 0), gb(b_ih, 1), gb(b_ih, 2),
          gb(b_hh, 0), gb(b_hh, 1), gb(b_hh, 2)]

    fwp = jnp.kron(eye8, jnp.zeros((16, 16), f32).at[:NF, :NO].set(fw.T))
    fbp = brow(jnp.zeros(16, f32).at[:NO].set(fb))

    h1_ = jnp.maximum(mb1[None, :], 0.0)
    h2_ = jnp.maximum(jnp.dot(h1_, mw2.T) + mb2[None, :], 0.0)
    h3_ = jnp.maximum(jnp.dot(h2_, mw3.T) + mb3[None, :], 0.0)
    m0 = jnp.dot(h3_, mw4.T) + mb4[None, :]
    m0t = jnp.zeros((128, 16), f32).at[:, :EF].set(m0)

    zrows = jnp.zeros((ROWS_PER_SUB, 16), f32)
    h128 = jnp.zeros((NP // 8, 128), f32)
    zagg = jnp.zeros((NP // 8, 128), f32)

    outs = []
    agg0 = _sc_scatter_const(dst2, m0t, zrows).reshape(NC, NP // 8, 128)
    aggs = (agg0[0], agg0[1])
    for it in range(ITERS):
        h128, o = _tc_gru(*aggs, ni128, h128, ws, bs, fwp, fbp)
        outs.append(o.reshape(NP, 16)[:N, :NO])
        if it < ITERS - 1:
            h16sc = h128.reshape(NP, 16)
            hs, hd = _sc_gather(h16sc, src, dst)
            msg = _tc_mlp(hs.reshape(EPH // 8, 128),
                          hd.reshape(EPH // 8, 128),
                          a1, b1, w2, w3, w4s, m1, m2, m3, m4)
            ap = _sc_scatter(dst2, msg.reshape(EPH, 16),
                             zrows).reshape(NC, NP // 8, 128)
            aggs = (ap[0], ap[1])
    return jnp.stack(outs, axis=0)
